# baseline (speedup 1.0000x reference)
import functools

import jax
import jax.numpy as jnp
from jax import lax
from jax.experimental import pallas as pl
from jax.experimental.pallas import tpu as pltpu
from jax.experimental.pallas import tpu_sc as plsc

_NC = 2
_NS = 16
_NW = _NC * _NS
_CHUNK = 16
_NBUF = 4


@functools.lru_cache(maxsize=None)
def _build_gather(N, V, D):
    n_per_w = N // _NW
    n_chunks = n_per_w // _CHUNK
    assert n_chunks % _NBUF == 0 and n_chunks >= 2 * _NBUF
    mesh = plsc.VectorSubcoreMesh(core_axis_name="c", subcore_axis_name="s")

    @functools.partial(
        pl.kernel,
        out_type=jax.ShapeDtypeStruct((N, D), jnp.float32),
        mesh=mesh,
        scratch_types=[
            pltpu.VMEM((n_chunks, _CHUNK), jnp.int32),
        ] + [pltpu.VMEM((_CHUNK, D), jnp.float32)] * _NBUF
          + [pltpu.SemaphoreType.DMA] * (2 * _NBUF),
    )
    def grab(idx_hbm, table_hbm, out_hbm, idx_v, b0, b1, b2, b3,
             g0, g1, g2, g3, w0, w1, w2, w3):
        wid = lax.axis_index("s") * _NC + lax.axis_index("c")
        base = wid * n_per_w
        pltpu.sync_copy(idx_hbm.at[wid], idx_v)
        bufs = (b0, b1, b2, b3)
        gsems = (g0, g1, g2, g3)
        wsems = (w0, w1, w2, w3)

        def g_start(g, b):
            pltpu.async_copy(table_hbm.at[idx_v.at[g]], bufs[b], gsems[b])

        def g_wait(g, b):
            pltpu.make_async_copy(
                table_hbm.at[idx_v.at[g]], bufs[b], gsems[b]).wait()

        def w_start(g, b):
            pltpu.async_copy(
                bufs[b], out_hbm.at[pl.ds(base + g * _CHUNK, _CHUNK)],
                wsems[b])

        def w_wait(g, b):
            pltpu.make_async_copy(
                bufs[b], out_hbm.at[pl.ds(base + g * _CHUNK, _CHUNK)],
                wsems[b]).wait()

        for g in range(_NBUF - 1):
            g_start(g, g)

        g_wait(0, 0)
        w_start(0, 0)
        g_start(_NBUF - 1, _NBUF - 1)

        def quad(p, carry):
            for j in range(_NBUF):
                s = 1 + _NBUF * p + j
                b = (1 + j) % _NBUF
                pb = (b + _NBUF - 1) % _NBUF
                g_wait(s, b)
                w_start(s, b)
                w_wait(s - 1, pb)
                g_start(s + _NBUF - 1, pb)
            return carry

        lax.fori_loop(0, (n_chunks - _NBUF) // _NBUF, quad, 0)

        for s in range(n_chunks - _NBUF + 1, n_chunks):
            b = s % _NBUF
            pb = (b + _NBUF - 1) % _NBUF
            g_wait(s, b)
            w_start(s, b)
            w_wait(s - 1, pb)
        w_wait(n_chunks - 1, (n_chunks - 1) % _NBUF)

    return grab


def kernel(positions, weight):
    B, S = positions.shape
    V, D = weight.shape
    N = B * S
    n_per_w = N // _NW
    idx = positions.astype(jnp.int32).reshape(_NW, n_per_w // _CHUNK, _CHUNK)
    out = _build_gather(N, V, D)(idx, weight)
    return out.reshape(B, S, D)
